# baseline (speedup 1.0000x reference)
import functools

import numpy as np
import jax
import jax.numpy as jnp
from jax import lax
from jax.experimental import pallas as pl
from jax.experimental.pallas import tpu as pltpu
from jax.experimental.pallas import tpu_sc as plsc

N = 50000
E = 800000
HID = 96
E_PAD = 802816
NB = 2000
NG = N // NB
EB = 4096
EG = E_PAD // EB
K = 128
NCHUNK = 4
CH_REAL = N // NCHUNK
CH = 12512
ROWS_PER_TILE = CH // 16


@functools.lru_cache(maxsize=None)
def _sc_mesh():
    return plsc.VectorSubcoreMesh(core_axis_name="c", subcore_axis_name="s")


@functools.lru_cache(maxsize=None)
def _make_gather(B, D):
    bw = B // 32
    nit = bw // K
    L = 4
    NBUF = 2 * L

    @functools.partial(
        pl.kernel, mesh=_sc_mesh(),
        compiler_params=pltpu.CompilerParams(use_tc_tiling_on_sc=False, skip_device_barrier=True),
        out_type=jax.ShapeDtypeStruct((B, D), jnp.float32),
        scratch_types=[
            pltpu.VMEM((nit, K), jnp.int32),
            pltpu.VMEM((NBUF, K, D), jnp.float32),
            pltpu.SemaphoreType.DMA((NBUF,)),
            pltpu.SemaphoreType.DMA((NBUF,)),
        ],
    )
    def gk(table, idx2, out, idx_t, bufs, gsem, wsem):
        wid = lax.axis_index("s") * 2 + lax.axis_index("c")
        base = wid * nit
        pltpu.sync_copy(idx2.at[pl.ds(base, nit)], idx_t)

        def g_start(j, b):
            pltpu.async_copy(table.at[idx_t.at[j]], bufs.at[b], gsem.at[b])

        def g_wait(j, b):
            pltpu.make_async_copy(
                table.at[idx_t.at[j]], bufs.at[b], gsem.at[b]).wait()

        def w_start(j, b):
            pltpu.async_copy(
                bufs.at[b], out.at[pl.ds((base + j) * K, K)], wsem.at[b])

        def w_wait(j, b):
            pltpu.make_async_copy(
                bufs.at[b], out.at[pl.ds((base + j) * K, K)], wsem.at[b]).wait()

        for j in range(L):
            g_start(j, j)
        for i in range(L):
            g_wait(i, i)
            w_start(i, i)
            g_start(i + L, i + L)

        def steady(i, carry):
            b = lax.rem(i, NBUF)
            g_wait(i, b)
            w_start(i, b)
            j = i + L
            bj = lax.rem(j, NBUF)
            w_wait(j - NBUF, bj)
            g_start(j, bj)
            return carry

        lax.fori_loop(L, nit - L, steady, 0)
        for i in range(nit - L, nit):
            g_wait(i, i % NBUF)
            w_start(i, i % NBUF)
        for j in range(nit - NBUF, nit):
            w_wait(j, j % NBUF)

    return gk


NBLK = 200
NBLOCKS = N // NBLK
CH_E = 1024


def _layer_kernel(off_ref, hs_hbm, ee_hbm, dc_hbm, h, w1s, w1d, w1e, eb1,
                  ew2, eb2, nw1h, nw1a, nb1, nw2, nb2, g, b, o_ref,
                  hs_b, ee_b, dc_b, sh, se, sd):
    nb = pl.program_id(0)
    lo = off_ref[nb]
    hi = off_ref[nb + 1]
    j0 = lo // CH_E
    j1 = lax.div(hi + CH_E - 1, CH_E)
    base = nb * NBLK
    hblk = h[...]
    hdp = _dot(hblk, w1d[...])

    def issue(j, bf):
        pltpu.make_async_copy(hs_hbm.at[pl.ds(j * CH_E, CH_E)],
                              hs_b.at[bf], sh.at[bf]).start()
        pltpu.make_async_copy(ee_hbm.at[pl.ds(j * CH_E, CH_E)],
                              ee_b.at[bf], se.at[bf]).start()
        pltpu.make_async_copy(dc_hbm.at[pl.ds(j * CH_E, CH_E)],
                              dc_b.at[bf], sd.at[bf]).start()

    def wait(j, bf):
        pltpu.make_async_copy(hs_hbm.at[pl.ds(j * CH_E, CH_E)],
                              hs_b.at[bf], sh.at[bf]).wait()
        pltpu.make_async_copy(ee_hbm.at[pl.ds(j * CH_E, CH_E)],
                              ee_b.at[bf], se.at[bf]).wait()
        pltpu.make_async_copy(dc_hbm.at[pl.ds(j * CH_E, CH_E)],
                              dc_b.at[bf], sd.at[bf]).wait()

    @pl.when(j0 < j1)
    def _():
        issue(j0, lax.rem(j0, 2))

    def body(j, carry):
        agg, cnt = carry
        bf = lax.rem(j, 2)
        wait(j, bf)

        @pl.when(j + 1 < j1)
        def _():
            issue(j + 1, lax.rem(j + 1, 2))

        d = dc_b[bf]
        pos = j * CH_E + lax.broadcasted_iota(jnp.int32, (CH_E, 1), 0)
        inr = (pos >= lo) & (pos < hi)
        lane = lax.broadcasted_iota(jnp.int32, (CH_E, NBLK), 1)
        oh = jnp.where((d - base == lane) & inr, 1.0, 0.0)
        z = (_dot(hs_b[bf], w1s[...]) + _dot(oh, hdp)
             + _dot(ee_b[bf], w1e[...]) + eb1[...])
        m = _dot(_gelu(z), ew2[...]) + eb2[...]
        agg = agg + _dot(oh.T, m)
        cnt = cnt + _dot(oh.T, jnp.ones((CH_E, 1), jnp.float32))
        return agg, cnt

    agg0 = jnp.zeros((NBLK, HID), jnp.float32)
    cnt0 = jnp.zeros((NBLK, 1), jnp.float32)
    agg, cnt = lax.fori_loop(j0, j1, body, (agg0, cnt0))
    a = agg / (cnt + 1e-8)
    z = _gelu(_dot(hblk, nw1h[...]) + _dot(a, nw1a[...]) + nb1[...])
    r = _dot(z, nw2[...]) + nb2[...] + hblk
    o_ref[...] = _ln(r, g[...], b[...])


_INV_SQRT2 = 0.7071067811865476


def _gelu(x):
    return 0.5 * x * (1.0 + lax.erf(x * _INV_SQRT2))


def _ln(x, g, b):
    mu = jnp.mean(x, axis=-1, keepdims=True)
    d = x - mu
    var = jnp.mean(d * d, axis=-1, keepdims=True)
    return d * lax.rsqrt(var + 1e-5) * g + b


def _dot(a, b):
    return jnp.dot(a, b, precision=lax.Precision.HIGHEST)


def _rep(shape):
    return pl.BlockSpec(shape, lambda i: (0, 0))


def _rep2(shape):
    return pl.BlockSpec(shape, lambda i, off: (0, 0))


def _mlp2_kernel(x_ref, w1, b1, w2, b2, o_ref):
    z = _gelu(_dot(x_ref[...], w1[...]) + b1[...])
    o_ref[...] = _dot(z, w2[...]) + b2[...]


def _mlp2_call(x, w1, b1, w2, b2, blk, grid_n, out_d):
    din = x.shape[1]
    return pl.pallas_call(
        _mlp2_kernel,
        grid=(grid_n,),
        in_specs=[
            pl.BlockSpec((blk, din), lambda i: (i, 0)),
            _rep(w1.shape), _rep(b1.shape), _rep(w2.shape), _rep(b2.shape),
        ],
        out_specs=pl.BlockSpec((blk, out_d), lambda i: (i, 0)),
        out_shape=jax.ShapeDtypeStruct((x.shape[0], out_d), jnp.float32),
    )(x, w1, b1, w2, b2)


def _ee_kernel(cs, cd, w1, b1, w2, b2, o_ref):
    d = cd[...] - cs[...]
    lane = lax.broadcasted_iota(jnp.int32, d.shape, 1)
    d3 = jnp.where(lane < 3, d, 0.0)
    dist = jnp.sqrt(jnp.sum(d3 * d3, axis=-1, keepdims=True))
    attr = d3 + jnp.where(lane == 3, dist, 0.0)
    z = _gelu(_dot(attr, w1[...]) + b1[...])
    o_ref[...] = _dot(z, w2[...]) + b2[...]


def _gt1_kernel(h, sqw, sqb, w_o, st_o):
    lg = _dot(h[...], sqw[...]) + sqb[...]
    lg = lg - jnp.max(lg, axis=-1, keepdims=True)
    ex = jnp.exp(lg)
    w = ex / jnp.sum(ex, axis=-1, keepdims=True)
    w_o[...] = w

    @pl.when(pl.program_id(0) == 0)
    def _():
        st_o[...] = jnp.zeros_like(st_o)

    st_o[...] += _dot(w.T, h[...])


def _gt2_kernel(st, in_w, in_b, out_w, out_b, f1, f1b, f2, f2b,
                g1, b1, g2, b2, o_ref):
    s = st[...]
    qkv = _dot(s, in_w[...]) + in_b[...]
    dh = HID // 8
    heads = []
    for hh in range(8):
        q = qkv[:, hh * dh:(hh + 1) * dh]
        k = qkv[:, HID + hh * dh:HID + (hh + 1) * dh]
        v = qkv[:, 2 * HID + hh * dh:2 * HID + (hh + 1) * dh]
        a = _dot(q, k.T) * (1.0 / np.sqrt(dh))
        a = a - jnp.max(a, axis=-1, keepdims=True)
        ea = jnp.exp(a)
        a = ea / jnp.sum(ea, axis=-1, keepdims=True)
        heads.append(_dot(a, v))
    o = jnp.concatenate(heads, axis=-1)
    o = _dot(o, out_w[...]) + out_b[...]
    s1 = _ln(s + o, g1[...], b1[...])
    ffn = _dot(_gelu(_dot(s1, f1[...]) + f1b[...]), f2[...]) + f2b[...]
    o_ref[...] = _ln(s1 + ffn, g2[...], b2[...])


def _gt3_kernel(w, st, h, o_ref):
    o_ref[...] = _dot(w[...], st[...]) + h[...]


def _dec_kernel(h, g, b, w1, b1, w2, b2, o_ref):
    hn = _ln(h[...], g[...], b[...])
    z = _gelu(_dot(hn, w1[...]) + b1[...])
    o_ref[...] = _dot(z, w2[...]) + b2[...]


def _row(v):
    return v.reshape(1, -1)


def kernel(x, coords, edge_index, params):
    src, dst = edge_index[0], edge_index[1]
    order = jnp.argsort(dst).astype(jnp.int32)
    pad = E_PAD - E
    zpad = jnp.zeros((pad,), jnp.int32)
    sd = jnp.concatenate(
        [src[:, None], dst[:, None], jnp.zeros((E, 14), jnp.int32)], axis=1)
    sdf = lax.bitcast_convert_type(sd, jnp.float32)
    ordp = jnp.concatenate([order, zpad]).reshape(E_PAD // K, K)
    g = lax.bitcast_convert_type(
        _make_gather(E_PAD, 16)(sdf, ordp), jnp.int32)
    epos = jnp.arange(E_PAD, dtype=jnp.int32)
    real = epos < E
    src_sp = jnp.where(real, g[:, 0], 0)
    dst_sp = g[:, 1]
    srcp = src_sp.reshape(E_PAD // K, K)
    dstg = jnp.where(real, dst_sp, 0).reshape(E_PAD // K, K)
    dc = jnp.where(real, dst_sp, N).reshape(E_PAD, 1)
    off = jnp.searchsorted(dst_sp[:E], jnp.arange(0, N + 1, NBLK)
                           ).astype(jnp.int32)
    coords16 = jnp.pad(coords, ((0, 0), (0, 16 - coords.shape[1])))
    x16 = jnp.pad(x, ((0, 0), (0, 16 - x.shape[1])))

    p = params

    cs = _make_gather(E_PAD, 16)(coords16, srcp)
    cd = _make_gather(E_PAD, 16)(coords16, dstg)
    ee_w1p = jnp.pad(p['ee_w1'], ((0, 16 - 4), (0, 0)))
    ee = pl.pallas_call(
        _ee_kernel,
        grid=(EG,),
        in_specs=[
            pl.BlockSpec((EB, 16), lambda i: (i, 0)),
            pl.BlockSpec((EB, 16), lambda i: (i, 0)),
            _rep((16, HID)), _rep((1, HID)), _rep((HID, HID)), _rep((1, HID)),
        ],
        out_specs=pl.BlockSpec((EB, HID), lambda i: (i, 0)),
        out_shape=jax.ShapeDtypeStruct((E_PAD, HID), jnp.float32),
    )(cs, cd, ee_w1p, _row(p['ee_b1']), p['ee_w2'], _row(p['ee_b2']))

    ne_w1p = jnp.pad(p['ne_w1'], ((0, 16 - 3), (0, 0)))
    h = _mlp2_call(x16, ne_w1p, _row(p['ne_b1']), p['ne_w2'],
                   _row(p['ne_b2']), NB, NG, HID)

    def mpnn(h, mp):
        hs = _make_gather(E_PAD, HID)(h, srcp)
        return pl.pallas_call(
            _layer_kernel,
            grid_spec=pltpu.PrefetchScalarGridSpec(
                num_scalar_prefetch=1,
                grid=(NBLOCKS,),
                in_specs=[
                    pl.BlockSpec(memory_space=pltpu.MemorySpace.HBM),
                    pl.BlockSpec(memory_space=pltpu.MemorySpace.HBM),
                    pl.BlockSpec(memory_space=pltpu.MemorySpace.HBM),
                    pl.BlockSpec((NBLK, HID), lambda i, off: (i, 0)),
                    _rep2((HID, HID)), _rep2((HID, HID)), _rep2((HID, HID)),
                    _rep2((1, HID)), _rep2((HID, HID)), _rep2((1, HID)),
                    _rep2((HID, HID)), _rep2((HID, HID)), _rep2((1, HID)),
                    _rep2((HID, HID)), _rep2((1, HID)),
                    _rep2((1, HID)), _rep2((1, HID)),
                ],
                out_specs=pl.BlockSpec((NBLK, HID), lambda i, off: (i, 0)),
                scratch_shapes=[
                    pltpu.VMEM((2, CH_E, HID), jnp.float32),
                    pltpu.VMEM((2, CH_E, HID), jnp.float32),
                    pltpu.VMEM((2, CH_E, 1), jnp.int32),
                    pltpu.SemaphoreType.DMA((2,)),
                    pltpu.SemaphoreType.DMA((2,)),
                    pltpu.SemaphoreType.DMA((2,)),
                ],
            ),
            out_shape=jax.ShapeDtypeStruct((N, HID), jnp.float32),
        )(off, hs, ee, dc, h,
          mp['e_w1'][0:HID], mp['e_w1'][HID:2 * HID],
          mp['e_w1'][2 * HID:3 * HID], _row(mp['e_b1']), mp['e_w2'],
          _row(mp['e_b2']),
          mp['n_w1'][0:HID], mp['n_w1'][HID:2 * HID], _row(mp['n_b1']),
          mp['n_w2'], _row(mp['n_b2']),
          _row(mp['ln_g']), _row(mp['ln_b']))

    for mp in p['pre']:
        h = mpnn(h, mp)

    gt = p['gt']
    w_all, st = pl.pallas_call(
        _gt1_kernel,
        grid=(NG,),
        in_specs=[
            pl.BlockSpec((NB, HID), lambda i: (i, 0)),
            _rep((HID, 32)), _rep((1, 32)),
        ],
        out_specs=[
            pl.BlockSpec((NB, 32), lambda i: (i, 0)),
            pl.BlockSpec((32, HID), lambda i: (0, 0)),
        ],
        out_shape=[
            jax.ShapeDtypeStruct((N, 32), jnp.float32),
            jax.ShapeDtypeStruct((32, HID), jnp.float32),
        ],
    )(h, gt['sq_w'], _row(gt['sq_b']))
    st = pl.pallas_call(
        _gt2_kernel,
        grid=(1,),
        in_specs=[_rep((32, HID)), _rep((HID, 3 * HID)), _rep((1, 3 * HID)),
                  _rep((HID, HID)), _rep((1, HID)),
                  _rep((HID, 4 * HID)), _rep((1, 4 * HID)),
                  _rep((4 * HID, HID)), _rep((1, HID)),
                  _rep((1, HID)), _rep((1, HID)), _rep((1, HID)),
                  _rep((1, HID))],
        out_specs=_rep((32, HID)),
        out_shape=jax.ShapeDtypeStruct((32, HID), jnp.float32),
    )(st, gt['in_w'], _row(gt['in_b']), gt['out_w'], _row(gt['out_b']),
      gt['ffn_w1'], _row(gt['ffn_b1']), gt['ffn_w2'], _row(gt['ffn_b2']),
      _row(gt['ln1_g']), _row(gt['ln1_b']), _row(gt['ln2_g']),
      _row(gt['ln2_b']))
    h = pl.pallas_call(
        _gt3_kernel,
        grid=(NG,),
        in_specs=[
            pl.BlockSpec((NB, 32), lambda i: (i, 0)),
            _rep((32, HID)),
            pl.BlockSpec((NB, HID), lambda i: (i, 0)),
        ],
        out_specs=pl.BlockSpec((NB, HID), lambda i: (i, 0)),
        out_shape=jax.ShapeDtypeStruct((N, HID), jnp.float32),
    )(w_all, st, h)

    for mp in p['post']:
        h = mpnn(h, mp)

    d = p['dec']
    return pl.pallas_call(
        _dec_kernel,
        grid=(NG,),
        in_specs=[
            pl.BlockSpec((NB, HID), lambda i: (i, 0)),
            _rep((1, HID)), _rep((1, HID)),
            _rep((HID, HID // 2)), _rep((1, HID // 2)),
            _rep((HID // 2, 9)), _rep((1, 9)),
        ],
        out_specs=pl.BlockSpec((NB, 9), lambda i: (i, 0)),
        out_shape=jax.ShapeDtypeStruct((N, 9), jnp.float32),
    )(h, _row(d['ln_g']), _row(d['ln_b']), d['w1'], _row(d['b1']),
      d['w2'], _row(d['b2']))


# speedup vs baseline: 1.0278x; 1.0278x over previous
import functools

import numpy as np
import jax
import jax.numpy as jnp
from jax import lax
from jax.experimental import pallas as pl
from jax.experimental.pallas import tpu as pltpu
from jax.experimental.pallas import tpu_sc as plsc

N = 50000
E = 800000
HID = 96
E_PAD = 802816
NB = 2000
NG = N // NB
EB = 4096
EG = E_PAD // EB
K = 128
NCHUNK = 4
CH_REAL = N // NCHUNK
CH = 12512
ROWS_PER_TILE = CH // 16


@functools.lru_cache(maxsize=None)
def _sc_mesh():
    return plsc.VectorSubcoreMesh(core_axis_name="c", subcore_axis_name="s")


@functools.lru_cache(maxsize=None)
def _make_gather(B, D):
    bw = B // 32
    nit = bw // K
    L = 4
    NBUF = 2 * L

    @functools.partial(
        pl.kernel, mesh=_sc_mesh(),
        compiler_params=pltpu.CompilerParams(use_tc_tiling_on_sc=False, skip_device_barrier=True),
        out_type=jax.ShapeDtypeStruct((B, D), jnp.float32),
        scratch_types=[
            pltpu.VMEM((nit, K), jnp.int32),
            pltpu.VMEM((NBUF, K, D), jnp.float32),
            pltpu.SemaphoreType.DMA((NBUF,)),
            pltpu.SemaphoreType.DMA((NBUF,)),
        ],
    )
    def gk(table, idx2, out, idx_t, bufs, gsem, wsem):
        wid = lax.axis_index("s") * 2 + lax.axis_index("c")
        base = wid * nit
        pltpu.sync_copy(idx2.at[pl.ds(base, nit)], idx_t)

        def g_start(j, b):
            pltpu.async_copy(table.at[idx_t.at[j]], bufs.at[b], gsem.at[b])

        def g_wait(j, b):
            pltpu.make_async_copy(
                table.at[idx_t.at[j]], bufs.at[b], gsem.at[b]).wait()

        def w_start(j, b):
            pltpu.async_copy(
                bufs.at[b], out.at[pl.ds((base + j) * K, K)], wsem.at[b])

        def w_wait(j, b):
            pltpu.make_async_copy(
                bufs.at[b], out.at[pl.ds((base + j) * K, K)], wsem.at[b]).wait()

        for j in range(L):
            g_start(j, j)
        for i in range(L):
            g_wait(i, i)
            w_start(i, i)
            g_start(i + L, i + L)

        def steady(i, carry):
            b = lax.rem(i, NBUF)
            g_wait(i, b)
            w_start(i, b)
            j = i + L
            bj = lax.rem(j, NBUF)
            w_wait(j - NBUF, bj)
            g_start(j, bj)
            return carry

        lax.fori_loop(L, nit - L, steady, 0)
        for i in range(nit - L, nit):
            g_wait(i, i % NBUF)
            w_start(i, i % NBUF)
        for j in range(nit - NBUF, nit):
            w_wait(j, j % NBUF)

    return gk


NBLK = 200
NBLOCKS = N // NBLK
CH_E = 1024


def _layer_kernel(off_ref, hs_hbm, ee_hbm, dc_hbm, h, w1s, w1d, w1e, eb1,
                  ew2, eb2, nw1h, nw1a, nb1, nw2, nb2, g, b, o_ref,
                  hs_b, ee_b, dc_b, sh, se, sd):
    nb = pl.program_id(0)
    lo = off_ref[nb]
    hi = off_ref[nb + 1]
    j0 = lo // CH_E
    j1 = lax.div(hi + CH_E - 1, CH_E)
    base = nb * NBLK
    hblk = h[...]
    hdp = _dot(hblk, w1d[...])

    def issue(j, bf):
        pltpu.make_async_copy(hs_hbm.at[pl.ds(j * CH_E, CH_E)],
                              hs_b.at[bf], sh.at[bf]).start()
        pltpu.make_async_copy(ee_hbm.at[pl.ds(j * CH_E, CH_E)],
                              ee_b.at[bf], se.at[bf]).start()
        pltpu.make_async_copy(dc_hbm.at[pl.ds(j * CH_E, CH_E)],
                              dc_b.at[bf], sd.at[bf]).start()

    def wait(j, bf):
        pltpu.make_async_copy(hs_hbm.at[pl.ds(j * CH_E, CH_E)],
                              hs_b.at[bf], sh.at[bf]).wait()
        pltpu.make_async_copy(ee_hbm.at[pl.ds(j * CH_E, CH_E)],
                              ee_b.at[bf], se.at[bf]).wait()
        pltpu.make_async_copy(dc_hbm.at[pl.ds(j * CH_E, CH_E)],
                              dc_b.at[bf], sd.at[bf]).wait()

    @pl.when(j0 < j1)
    def _():
        issue(j0, lax.rem(j0, 2))

    def body(j, carry):
        agg, cnt = carry
        bf = lax.rem(j, 2)
        wait(j, bf)

        @pl.when(j + 1 < j1)
        def _():
            issue(j + 1, lax.rem(j + 1, 2))

        d = dc_b[bf]
        pos = j * CH_E + lax.broadcasted_iota(jnp.int32, (CH_E, 1), 0)
        inr = (pos >= lo) & (pos < hi)
        lane = lax.broadcasted_iota(jnp.int32, (CH_E, NBLK), 1)
        oh = jnp.where((d - base == lane) & inr, 1.0, 0.0)
        z = (_dot(hs_b[bf], w1s[...]) + _dot(oh, hdp)
             + _dot(ee_b[bf], w1e[...]) + eb1[...])
        m = _dot(_gelu(z), ew2[...]) + eb2[...]
        agg = agg + _dot(oh.T, m)
        cnt = cnt + _dot(oh.T, jnp.ones((CH_E, 1), jnp.float32))
        return agg, cnt

    agg0 = jnp.zeros((NBLK, HID), jnp.float32)
    cnt0 = jnp.zeros((NBLK, 1), jnp.float32)
    agg, cnt = lax.fori_loop(j0, j1, body, (agg0, cnt0))
    a = agg / (cnt + 1e-8)
    z = _gelu(_dot(hblk, nw1h[...]) + _dot(a, nw1a[...]) + nb1[...])
    r = _dot(z, nw2[...]) + nb2[...] + hblk
    o_ref[...] = _ln(r, g[...], b[...])


_INV_SQRT2 = 0.7071067811865476


def _gelu(x):
    return 0.5 * x * (1.0 + lax.erf(x * _INV_SQRT2))


def _ln(x, g, b):
    mu = jnp.mean(x, axis=-1, keepdims=True)
    d = x - mu
    var = jnp.mean(d * d, axis=-1, keepdims=True)
    return d * lax.rsqrt(var + 1e-5) * g + b


def _dot(a, b):
    return jnp.dot(a, b, precision=lax.Precision.HIGHEST)


def _rep(shape):
    return pl.BlockSpec(shape, lambda i: (0, 0))


def _rep2(shape):
    return pl.BlockSpec(shape, lambda i, off: (0, 0))


def _mlp2_kernel(x_ref, w1, b1, w2, b2, o_ref):
    z = _gelu(_dot(x_ref[...], w1[...]) + b1[...])
    o_ref[...] = _dot(z, w2[...]) + b2[...]


def _mlp2_call(x, w1, b1, w2, b2, blk, grid_n, out_d):
    din = x.shape[1]
    return pl.pallas_call(
        _mlp2_kernel,
        grid=(grid_n,),
        in_specs=[
            pl.BlockSpec((blk, din), lambda i: (i, 0)),
            _rep(w1.shape), _rep(b1.shape), _rep(w2.shape), _rep(b2.shape),
        ],
        out_specs=pl.BlockSpec((blk, out_d), lambda i: (i, 0)),
        out_shape=jax.ShapeDtypeStruct((x.shape[0], out_d), jnp.float32),
    )(x, w1, b1, w2, b2)


def _ee_kernel(cs, cd, w1, b1, w2, b2, o_ref):
    d = cd[...] - cs[...]
    lane = lax.broadcasted_iota(jnp.int32, d.shape, 1)
    d3 = jnp.where(lane < 3, d, 0.0)
    dist = jnp.sqrt(jnp.sum(d3 * d3, axis=-1, keepdims=True))
    attr = d3 + jnp.where(lane == 3, dist, 0.0)
    z = _gelu(_dot(attr, w1[...]) + b1[...])
    o_ref[...] = _dot(z, w2[...]) + b2[...]


def _gt1_kernel(h, sqw, sqb, w_o, st_o):
    lg = _dot(h[...], sqw[...]) + sqb[...]
    lg = lg - jnp.max(lg, axis=-1, keepdims=True)
    ex = jnp.exp(lg)
    w = ex / jnp.sum(ex, axis=-1, keepdims=True)
    w_o[...] = w

    @pl.when(pl.program_id(0) == 0)
    def _():
        st_o[...] = jnp.zeros_like(st_o)

    st_o[...] += _dot(w.T, h[...])


def _gt2_kernel(st, in_w, in_b, out_w, out_b, f1, f1b, f2, f2b,
                g1, b1, g2, b2, o_ref):
    s = st[...]
    qkv = _dot(s, in_w[...]) + in_b[...]
    dh = HID // 8
    heads = []
    for hh in range(8):
        q = qkv[:, hh * dh:(hh + 1) * dh]
        k = qkv[:, HID + hh * dh:HID + (hh + 1) * dh]
        v = qkv[:, 2 * HID + hh * dh:2 * HID + (hh + 1) * dh]
        a = _dot(q, k.T) * (1.0 / np.sqrt(dh))
        a = a - jnp.max(a, axis=-1, keepdims=True)
        ea = jnp.exp(a)
        a = ea / jnp.sum(ea, axis=-1, keepdims=True)
        heads.append(_dot(a, v))
    o = jnp.concatenate(heads, axis=-1)
    o = _dot(o, out_w[...]) + out_b[...]
    s1 = _ln(s + o, g1[...], b1[...])
    ffn = _dot(_gelu(_dot(s1, f1[...]) + f1b[...]), f2[...]) + f2b[...]
    o_ref[...] = _ln(s1 + ffn, g2[...], b2[...])


def _gt3_kernel(w, st, h, o_ref):
    o_ref[...] = _dot(w[...], st[...]) + h[...]


def _dec_kernel(h, g, b, w1, b1, w2, b2, o_ref):
    hn = _ln(h[...], g[...], b[...])
    z = _gelu(_dot(hn, w1[...]) + b1[...])
    o_ref[...] = _dot(z, w2[...]) + b2[...]


def _row(v):
    return v.reshape(1, -1)


def kernel(x, coords, edge_index, params):
    src, dst = edge_index[0], edge_index[1]
    order = jnp.argsort(dst).astype(jnp.int32)
    src_s = src[order]
    dst_s = dst[order]
    pad = E_PAD - E
    zpad = jnp.zeros((pad,), jnp.int32)
    srcp = jnp.concatenate([src_s, zpad]).reshape(E_PAD // K, K)
    dstg = jnp.concatenate([dst_s, zpad]).reshape(E_PAD // K, K)
    dc = jnp.concatenate([dst_s, jnp.full((pad,), N, jnp.int32)]
                         ).reshape(E_PAD, 1)
    off = jnp.searchsorted(dst_s, jnp.arange(0, N + 1, NBLK)
                           ).astype(jnp.int32)
    coords16 = jnp.pad(coords, ((0, 0), (0, 16 - coords.shape[1])))
    x16 = jnp.pad(x, ((0, 0), (0, 16 - x.shape[1])))

    p = params

    cs = _make_gather(E_PAD, 16)(coords16, srcp)
    cd = _make_gather(E_PAD, 16)(coords16, dstg)
    ee_w1p = jnp.pad(p['ee_w1'], ((0, 16 - 4), (0, 0)))
    ee = pl.pallas_call(
        _ee_kernel,
        grid=(EG,),
        in_specs=[
            pl.BlockSpec((EB, 16), lambda i: (i, 0)),
            pl.BlockSpec((EB, 16), lambda i: (i, 0)),
            _rep((16, HID)), _rep((1, HID)), _rep((HID, HID)), _rep((1, HID)),
        ],
        out_specs=pl.BlockSpec((EB, HID), lambda i: (i, 0)),
        out_shape=jax.ShapeDtypeStruct((E_PAD, HID), jnp.float32),
    )(cs, cd, ee_w1p, _row(p['ee_b1']), p['ee_w2'], _row(p['ee_b2']))

    ne_w1p = jnp.pad(p['ne_w1'], ((0, 16 - 3), (0, 0)))
    h = _mlp2_call(x16, ne_w1p, _row(p['ne_b1']), p['ne_w2'],
                   _row(p['ne_b2']), NB, NG, HID)

    def mpnn(h, mp):
        hs = _make_gather(E_PAD, HID)(h, srcp)
        return pl.pallas_call(
            _layer_kernel,
            grid_spec=pltpu.PrefetchScalarGridSpec(
                num_scalar_prefetch=1,
                grid=(NBLOCKS,),
                in_specs=[
                    pl.BlockSpec(memory_space=pltpu.MemorySpace.HBM),
                    pl.BlockSpec(memory_space=pltpu.MemorySpace.HBM),
                    pl.BlockSpec(memory_space=pltpu.MemorySpace.HBM),
                    pl.BlockSpec((NBLK, HID), lambda i, off: (i, 0)),
                    _rep2((HID, HID)), _rep2((HID, HID)), _rep2((HID, HID)),
                    _rep2((1, HID)), _rep2((HID, HID)), _rep2((1, HID)),
                    _rep2((HID, HID)), _rep2((HID, HID)), _rep2((1, HID)),
                    _rep2((HID, HID)), _rep2((1, HID)),
                    _rep2((1, HID)), _rep2((1, HID)),
                ],
                out_specs=pl.BlockSpec((NBLK, HID), lambda i, off: (i, 0)),
                scratch_shapes=[
                    pltpu.VMEM((2, CH_E, HID), jnp.float32),
                    pltpu.VMEM((2, CH_E, HID), jnp.float32),
                    pltpu.VMEM((2, CH_E, 1), jnp.int32),
                    pltpu.SemaphoreType.DMA((2,)),
                    pltpu.SemaphoreType.DMA((2,)),
                    pltpu.SemaphoreType.DMA((2,)),
                ],
            ),
            out_shape=jax.ShapeDtypeStruct((N, HID), jnp.float32),
        )(off, hs, ee, dc, h,
          mp['e_w1'][0:HID], mp['e_w1'][HID:2 * HID],
          mp['e_w1'][2 * HID:3 * HID], _row(mp['e_b1']), mp['e_w2'],
          _row(mp['e_b2']),
          mp['n_w1'][0:HID], mp['n_w1'][HID:2 * HID], _row(mp['n_b1']),
          mp['n_w2'], _row(mp['n_b2']),
          _row(mp['ln_g']), _row(mp['ln_b']))

    for mp in p['pre']:
        h = mpnn(h, mp)

    gt = p['gt']
    w_all, st = pl.pallas_call(
        _gt1_kernel,
        grid=(NG,),
        in_specs=[
            pl.BlockSpec((NB, HID), lambda i: (i, 0)),
            _rep((HID, 32)), _rep((1, 32)),
        ],
        out_specs=[
            pl.BlockSpec((NB, 32), lambda i: (i, 0)),
            pl.BlockSpec((32, HID), lambda i: (0, 0)),
        ],
        out_shape=[
            jax.ShapeDtypeStruct((N, 32), jnp.float32),
            jax.ShapeDtypeStruct((32, HID), jnp.float32),
        ],
    )(h, gt['sq_w'], _row(gt['sq_b']))
    st = pl.pallas_call(
        _gt2_kernel,
        grid=(1,),
        in_specs=[_rep((32, HID)), _rep((HID, 3 * HID)), _rep((1, 3 * HID)),
                  _rep((HID, HID)), _rep((1, HID)),
                  _rep((HID, 4 * HID)), _rep((1, 4 * HID)),
                  _rep((4 * HID, HID)), _rep((1, HID)),
                  _rep((1, HID)), _rep((1, HID)), _rep((1, HID)),
                  _rep((1, HID))],
        out_specs=_rep((32, HID)),
        out_shape=jax.ShapeDtypeStruct((32, HID), jnp.float32),
    )(st, gt['in_w'], _row(gt['in_b']), gt['out_w'], _row(gt['out_b']),
      gt['ffn_w1'], _row(gt['ffn_b1']), gt['ffn_w2'], _row(gt['ffn_b2']),
      _row(gt['ln1_g']), _row(gt['ln1_b']), _row(gt['ln2_g']),
      _row(gt['ln2_b']))
    h = pl.pallas_call(
        _gt3_kernel,
        grid=(NG,),
        in_specs=[
            pl.BlockSpec((NB, 32), lambda i: (i, 0)),
            _rep((32, HID)),
            pl.BlockSpec((NB, HID), lambda i: (i, 0)),
        ],
        out_specs=pl.BlockSpec((NB, HID), lambda i: (i, 0)),
        out_shape=jax.ShapeDtypeStruct((N, HID), jnp.float32),
    )(w_all, st, h)

    for mp in p['post']:
        h = mpnn(h, mp)

    d = p['dec']
    return pl.pallas_call(
        _dec_kernel,
        grid=(NG,),
        in_specs=[
            pl.BlockSpec((NB, HID), lambda i: (i, 0)),
            _rep((1, HID)), _rep((1, HID)),
            _rep((HID, HID // 2)), _rep((1, HID // 2)),
            _rep((HID // 2, 9)), _rep((1, 9)),
        ],
        out_specs=pl.BlockSpec((NB, 9), lambda i: (i, 0)),
        out_shape=jax.ShapeDtypeStruct((N, 9), jnp.float32),
    )(h, _row(d['ln_g']), _row(d['ln_b']), d['w1'], _row(d['b1']),
      d['w2'], _row(d['b2']))
